# baseline (speedup 1.0000x reference)
import functools

import jax
import jax.numpy as jnp
from jax import lax
from jax.experimental import pallas as pl
from jax.experimental.pallas import tpu as pltpu
from jax.experimental.pallas import tpu_sc as plsc

_LANES = 16


def _pick_chunk(ept: int) -> int:
    for c in range(128, 0, -8):
        if ept % c == 0:
            return c
    raise ValueError(f"no chunk size divides {ept}")


def _sc_aggregate(x, src3, dst, edge_attrs):
    N, D = x.shape
    E = edge_attrs.shape[0]
    info = plsc.get_sparse_core_info()
    NC, NS = info.num_cores, info.num_subcores
    NW = NC * NS
    assert E % NW == 0 and N % NS == 0 and D % _LANES == 0
    EPT = E // NW
    NW_, NCHUNK, C = src3.shape
    assert NW_ == NW and NCHUNK * C == EPT
    RPT = (N // NS) // 8 * 8
    REM = N - NS * RPT
    assert REM % 8 == 0 and REM <= C
    ZFULL, ZREM = RPT // C, RPT % C

    mesh = plsc.VectorSubcoreMesh(core_axis_name="c", subcore_axis_name="s")

    @functools.partial(
        pl.kernel,
        out_type=jax.ShapeDtypeStruct((NC, N, D), jnp.float32),
        mesh=mesh,
        scratch_types=[
            pltpu.VMEM((NCHUNK, C), jnp.int32),
            pltpu.VMEM((C // 2,), jnp.int32),
            pltpu.VMEM((C // 2,), jnp.int32),
            pltpu.VMEM((C, D), jnp.float32),
            pltpu.VMEM((C, D), jnp.float32),
            pltpu.VMEM_SHARED((N, D), jnp.float32),
            pltpu.SemaphoreType.DMA,
            pltpu.SemaphoreType.DMA,
            pltpu.SemaphoreType.DMA,
            pltpu.SemaphoreType.DMA,
            pltpu.SemaphoreType.DMA,
            pltpu.SemaphoreType.DMA,
            pltpu.SemaphoreType.DMA,
            pltpu.SemaphoreType.DMA,
            pltpu.SemaphoreType.DMA,
        ],
    )
    def agg_kernel(x_hbm, src_hbm, dst_hbm, ea_hbm, out_hbm,
                   src_2d, dva, dvb, ea_v, xr_v, acc_sh, sem, sem2, sem3,
                   sem4, sem5, sem6, sem7, sem8, sem9):
        del sem5
        c = lax.axis_index("c")
        s = lax.axis_index("s")
        wid = c * NS + s
        row0 = s * RPT

        pltpu.sync_copy(src_hbm.at[wid], src_2d)

        def zrow(e, carry):
            for j in range(D // _LANES):
                ea_v[e, pl.ds(j * _LANES, _LANES)] = jnp.zeros(
                    (_LANES,), jnp.float32)
            return carry
        lax.fori_loop(0, C, zrow, 0)
        for k in range(ZFULL):
            pltpu.sync_copy(ea_v, acc_sh.at[pl.ds(row0 + k * C, C)])
        if ZREM:
            pltpu.sync_copy(ea_v.at[pl.ds(0, ZREM)],
                            acc_sh.at[pl.ds(row0 + ZFULL * C, ZREM)])
        if REM:
            @pl.when(s == NS - 1)
            def _zero_tail():
                pltpu.sync_copy(ea_v.at[pl.ds(0, REM)],
                                acc_sh.at[pl.ds(NS * RPT, REM)])
        plsc.subcore_barrier()

        ebase = wid * EPT

        H = C // 2
        assert H % 8 == 0

        def chunk(i, carry):
            b = ebase + i * C
            pltpu.async_copy(x_hbm.at[src_2d.at[i, pl.ds(0, H)]],
                             xr_v.at[pl.ds(0, H)], sem)
            pltpu.async_copy(ea_hbm.at[pl.ds(b, H)],
                             ea_v.at[pl.ds(0, H)], sem2)
            pltpu.async_copy(x_hbm.at[src_2d.at[i, pl.ds(H, H)]],
                             xr_v.at[pl.ds(H, H)], sem4)
            pltpu.async_copy(ea_hbm.at[pl.ds(b + H, H)],
                             ea_v.at[pl.ds(H, H)], sem3)
            pltpu.async_copy(dst_hbm.at[pl.ds(b, H)], dva, sem6)
            pltpu.async_copy(dst_hbm.at[pl.ds(b + H, H)], dvb, sem7)

            def edge(e, carry2):
                for j in range(D // _LANES):
                    sl = pl.ds(j * _LANES, _LANES)
                    ea_v[e, sl] = jnp.maximum(ea_v[e, sl] + xr_v[e, sl], 0.0)
                return carry2

            pltpu.make_async_copy(ea_hbm.at[pl.ds(b, H)],
                                  ea_v.at[pl.ds(0, H)], sem2).wait()
            pltpu.make_async_copy(x_hbm.at[src_2d.at[i, pl.ds(0, H)]],
                                  xr_v.at[pl.ds(0, H)], sem).wait()
            lax.fori_loop(0, H, edge, 0, unroll=2)
            pltpu.make_async_copy(dst_hbm.at[pl.ds(b, H)], dva,
                                  sem6).wait()
            pltpu.async_copy(ea_v.at[pl.ds(0, H)], acc_sh.at[dva],
                             sem8, add=True)
            pltpu.make_async_copy(ea_hbm.at[pl.ds(b + H, H)],
                                  ea_v.at[pl.ds(H, H)], sem3).wait()
            pltpu.make_async_copy(x_hbm.at[src_2d.at[i, pl.ds(H, H)]],
                                  xr_v.at[pl.ds(H, H)], sem4).wait()
            lax.fori_loop(H, C, edge, 0, unroll=2)
            pltpu.make_async_copy(dst_hbm.at[pl.ds(b + H, H)], dvb,
                                  sem7).wait()
            pltpu.async_copy(ea_v.at[pl.ds(H, H)], acc_sh.at[dvb],
                             sem9, add=True)
            pltpu.make_async_copy(ea_v.at[pl.ds(0, H)], acc_sh.at[dva],
                                  sem8).wait()
            pltpu.make_async_copy(ea_v.at[pl.ds(H, H)], acc_sh.at[dvb],
                                  sem9).wait()
            return carry
        lax.fori_loop(0, NCHUNK, chunk, 0)

        plsc.subcore_barrier()
        pltpu.sync_copy(acc_sh.at[pl.ds(row0, RPT)],
                        out_hbm.at[c, pl.ds(row0, RPT)])
        if REM:
            @pl.when(s == NS - 1)
            def _drain_tail():
                pltpu.sync_copy(acc_sh.at[pl.ds(NS * RPT, REM)],
                                out_hbm.at[c, pl.ds(NS * RPT, REM)])

    return agg_kernel(x, src3, dst, edge_attrs)


def _tc_layer(x, agg, W, b):
    N, D = x.shape
    R = 1000 if N % 1000 == 0 else N
    grid = N // R

    def body(x_ref, a0_ref, a1_ref, w_ref, b_ref, o_ref):
        ssum = x_ref[...] + a0_ref[...] + a1_ref[...]
        o_ref[...] = jnp.maximum(
            jnp.dot(ssum, w_ref[...], preferred_element_type=jnp.float32)
            + b_ref[...], 0.0)

    return pl.pallas_call(
        body,
        grid=(grid,),
        in_specs=[
            pl.BlockSpec((R, D), lambda i: (i, 0)),
            pl.BlockSpec((R, D), lambda i: (i, 0)),
            pl.BlockSpec((R, D), lambda i: (i, 0)),
            pl.BlockSpec((D, D), lambda i: (0, 0)),
            pl.BlockSpec((1, D), lambda i: (0, 0)),
        ],
        out_specs=pl.BlockSpec((R, D), lambda i: (i, 0)),
        out_shape=jax.ShapeDtypeStruct((N, D), jnp.float32),
    )(x, agg[0], agg[1], W, b.reshape(1, D))


def kernel(node_feats, edge_index, edge_attrs, W1, b1, W2, b2):
    E = edge_attrs.shape[0]
    info = plsc.get_sparse_core_info()
    NW = info.num_cores * info.num_subcores
    EPT = E // NW
    C = _pick_chunk(EPT)
    src3 = edge_index[0].astype(jnp.int32).reshape(NW, EPT // C, C)
    dst = edge_index[1].astype(jnp.int32)
    agg1 = _sc_aggregate(node_feats, src3, dst, edge_attrs)
    h1 = _tc_layer(node_feats, agg1, W1, b1)
    agg2 = _sc_aggregate(h1, src3, dst, edge_attrs)
    h2 = _tc_layer(h1, agg2, W2, b2)
    return h2


# speedup vs baseline: 1.9371x; 1.9371x over previous
import functools

import jax
import jax.numpy as jnp
from jax import lax
from jax.experimental import pallas as pl
from jax.experimental.pallas import tpu as pltpu
from jax.experimental.pallas import tpu_sc as plsc

_LANES = 16


def _pick_chunk(ept: int) -> int:
    for c in range(128, 0, -8):
        if ept % c == 0:
            return c
    raise ValueError(f"no chunk size divides {ept}")


def _sc_aggregate(x, src3, dst, edge_attrs):
    N, D = x.shape
    E = edge_attrs.shape[0]
    info = plsc.get_sparse_core_info()
    NC, NS = info.num_cores, info.num_subcores
    NW = NC * NS
    assert E % NW == 0 and N % NS == 0 and D % _LANES == 0
    EPT = E // NW
    NW_, NCHUNK, C = src3.shape
    assert NW_ == NW and NCHUNK * C == EPT
    RPT = (N // NS) // 8 * 8
    REM = N - NS * RPT
    assert REM % 8 == 0 and REM <= C
    ZFULL, ZREM = RPT // C, RPT % C

    mesh = plsc.VectorSubcoreMesh(core_axis_name="c", subcore_axis_name="s")

    @functools.partial(
        pl.kernel,
        out_type=jax.ShapeDtypeStruct((NC, N, D), jnp.float32),
        mesh=mesh,
        scratch_types=[
            pltpu.VMEM((NCHUNK, C), jnp.int32),
            pltpu.VMEM((C // 2,), jnp.int32),
            pltpu.VMEM((C // 2,), jnp.int32),
            pltpu.VMEM((C, D), jnp.float32),
            pltpu.VMEM((C, D), jnp.float32),
            pltpu.VMEM_SHARED((N, D), jnp.float32),
            pltpu.SemaphoreType.DMA,
            pltpu.SemaphoreType.DMA,
            pltpu.SemaphoreType.DMA,
            pltpu.SemaphoreType.DMA,
            pltpu.SemaphoreType.DMA,
            pltpu.SemaphoreType.DMA,
            pltpu.SemaphoreType.DMA,
            pltpu.SemaphoreType.DMA,
            pltpu.SemaphoreType.DMA,
        ],
    )
    def agg_kernel(x_hbm, src_hbm, dst_hbm, ea_hbm, out_hbm,
                   src_2d, dva, dvb, ea_v, xr_v, acc_sh, sem, sem2, sem3,
                   sem4, sem5, sem6, sem7, sem8, sem9):
        del sem5
        c = lax.axis_index("c")
        s = lax.axis_index("s")
        wid = c * NS + s
        row0 = s * RPT

        pltpu.sync_copy(src_hbm.at[wid], src_2d)

        def zrow(e, carry):
            for j in range(D // _LANES):
                ea_v[e, pl.ds(j * _LANES, _LANES)] = jnp.zeros(
                    (_LANES,), jnp.float32)
            return carry
        lax.fori_loop(0, C, zrow, 0)
        for k in range(ZFULL):
            pltpu.sync_copy(ea_v, acc_sh.at[pl.ds(row0 + k * C, C)])
        if ZREM:
            pltpu.sync_copy(ea_v.at[pl.ds(0, ZREM)],
                            acc_sh.at[pl.ds(row0 + ZFULL * C, ZREM)])
        if REM:
            @pl.when(s == NS - 1)
            def _zero_tail():
                pltpu.sync_copy(ea_v.at[pl.ds(0, REM)],
                                acc_sh.at[pl.ds(NS * RPT, REM)])
        plsc.subcore_barrier()

        ebase = wid * EPT

        H = C // 2
        assert H % 8 == 0

        def chunk(i, carry):
            b = ebase + i * C
            pltpu.async_copy(x_hbm.at[src_2d.at[i, pl.ds(0, H)]],
                             xr_v.at[pl.ds(0, H)], sem)
            pltpu.async_copy(x_hbm.at[src_2d.at[i, pl.ds(H, H)]],
                             xr_v.at[pl.ds(H, H)], sem4)
            pltpu.async_copy(dst_hbm.at[pl.ds(b, H)], dva, sem6)
            pltpu.async_copy(dst_hbm.at[pl.ds(b + H, H)], dvb, sem7)
            pltpu.async_copy(ea_hbm.at[pl.ds(b, H)],
                             ea_v.at[pl.ds(0, H)], sem2)
            pltpu.async_copy(ea_hbm.at[pl.ds(b + H, H)],
                             ea_v.at[pl.ds(H, H)], sem3)

            def edge(e, carry2):
                for j in range(D // _LANES):
                    sl = pl.ds(j * _LANES, _LANES)
                    ea_v[e, sl] = jnp.maximum(ea_v[e, sl] + xr_v[e, sl], 0.0)
                return carry2

            pltpu.make_async_copy(ea_hbm.at[pl.ds(b, H)],
                                  ea_v.at[pl.ds(0, H)], sem2).wait()
            pltpu.make_async_copy(x_hbm.at[src_2d.at[i, pl.ds(0, H)]],
                                  xr_v.at[pl.ds(0, H)], sem).wait()
            lax.fori_loop(0, H, edge, 0)
            pltpu.make_async_copy(dst_hbm.at[pl.ds(b, H)], dva,
                                  sem6).wait()
            pltpu.async_copy(ea_v.at[pl.ds(0, H)], acc_sh.at[dva],
                             sem8, add=True)
            pltpu.make_async_copy(ea_hbm.at[pl.ds(b + H, H)],
                                  ea_v.at[pl.ds(H, H)], sem3).wait()
            pltpu.make_async_copy(x_hbm.at[src_2d.at[i, pl.ds(H, H)]],
                                  xr_v.at[pl.ds(H, H)], sem4).wait()
            lax.fori_loop(H, C, edge, 0)
            pltpu.make_async_copy(dst_hbm.at[pl.ds(b + H, H)], dvb,
                                  sem7).wait()
            pltpu.async_copy(ea_v.at[pl.ds(H, H)], acc_sh.at[dvb],
                             sem9, add=True)
            pltpu.make_async_copy(ea_v.at[pl.ds(0, H)], acc_sh.at[dva],
                                  sem8).wait()
            pltpu.make_async_copy(ea_v.at[pl.ds(H, H)], acc_sh.at[dvb],
                                  sem9).wait()
            return carry
        lax.fori_loop(0, NCHUNK, chunk, 0)

        plsc.subcore_barrier()
        pltpu.sync_copy(acc_sh.at[pl.ds(row0, RPT)],
                        out_hbm.at[c, pl.ds(row0, RPT)])
        if REM:
            @pl.when(s == NS - 1)
            def _drain_tail():
                pltpu.sync_copy(acc_sh.at[pl.ds(NS * RPT, REM)],
                                out_hbm.at[c, pl.ds(NS * RPT, REM)])

    return agg_kernel(x, src3, dst, edge_attrs)


def _tc_layer(x, agg, W, b):
    N, D = x.shape
    R = 1000 if N % 1000 == 0 else N
    grid = N // R

    def body(x_ref, a0_ref, a1_ref, w_ref, b_ref, o_ref):
        ssum = x_ref[...] + a0_ref[...] + a1_ref[...]
        o_ref[...] = jnp.maximum(
            jnp.dot(ssum, w_ref[...], preferred_element_type=jnp.float32)
            + b_ref[...], 0.0)

    return pl.pallas_call(
        body,
        grid=(grid,),
        in_specs=[
            pl.BlockSpec((R, D), lambda i: (i, 0)),
            pl.BlockSpec((R, D), lambda i: (i, 0)),
            pl.BlockSpec((R, D), lambda i: (i, 0)),
            pl.BlockSpec((D, D), lambda i: (0, 0)),
            pl.BlockSpec((1, D), lambda i: (0, 0)),
        ],
        out_specs=pl.BlockSpec((R, D), lambda i: (i, 0)),
        out_shape=jax.ShapeDtypeStruct((N, D), jnp.float32),
    )(x, agg[0], agg[1], W, b.reshape(1, D))


def kernel(node_feats, edge_index, edge_attrs, W1, b1, W2, b2):
    E = edge_attrs.shape[0]
    info = plsc.get_sparse_core_info()
    NW = info.num_cores * info.num_subcores
    EPT = E // NW
    C = _pick_chunk(EPT)
    src3 = edge_index[0].astype(jnp.int32).reshape(NW, EPT // C, C)
    dst = edge_index[1].astype(jnp.int32)
    agg1 = _sc_aggregate(node_feats, src3, dst, edge_attrs)
    h1 = _tc_layer(node_feats, agg1, W1, b1)
    agg2 = _sc_aggregate(h1, src3, dst, edge_attrs)
    h2 = _tc_layer(h1, agg2, W2, b2)
    return h2
